# baseline (speedup 1.0000x reference)
import functools

import jax
import jax.numpy as jnp
from jax import lax
from jax.experimental import pallas as pl
from jax.experimental.pallas import tpu as pltpu
from jax.experimental.pallas import tpu_sc as plsc

N = 100000
E = 3200000
NC, NS = 2, 16
NW = NC * NS
TW = 48
GB = 800
SB = 800
EW = E // NW
ES = E // NS

_mesh = plsc.VectorSubcoreMesh(core_axis_name="c", subcore_axis_name="s")
_sc_params = pltpu.CompilerParams(use_tc_tiling_on_sc=False)


def _gather_body(p_hbm, q_hbm, i_hbm, j_hbm, h_hbm, r_hbm, ib, jb, pb, qb, hb,
                 rb, s1, s2):
    cid = lax.axis_index("c")
    sid = lax.axis_index("s")
    base = (sid * NC + cid) * EW

    def zrow(q, cc):
        for r in range(4):
            rb[q, pl.ds(32 * r + 16, 16)] = jnp.zeros((16,), jnp.float32)
        return cc

    lax.fori_loop(0, GB // 4, zrow, 0)

    def chunk(c, carry):
        e0 = base + c * GB
        pltpu.sync_copy(i_hbm.at[pl.ds(e0, GB)], ib)
        pltpu.sync_copy(j_hbm.at[pl.ds(e0, GB)], jb)
        cp1 = pltpu.async_copy(p_hbm.at[ib], pb, s1)
        cp2 = pltpu.async_copy(q_hbm.at[jb], qb, s2)
        cp1.wait()
        cp2.wait()

        def add_q(q, c2):
            for r in range(4):
                e = 4 * q + r
                for g in range(2):
                    hb[q, pl.ds(32 * r + 16 * g, 16)] = (
                        pb[e, pl.ds(16 * g, 16)] + qb[e, pl.ds(16 * g, 16)])
                rb[q, pl.ds(32 * r, 16)] = (
                    pb[e, pl.ds(32, 16)] + qb[e, pl.ds(32, 16)])
            return c2

        lax.fori_loop(0, GB // 4, add_q, 0)
        pltpu.sync_copy(hb, h_hbm.at[pl.ds(e0 // 4, GB // 4)])
        pltpu.sync_copy(rb, r_hbm.at[pl.ds(e0 // 4, GB // 4)])
        return carry

    lax.fori_loop(0, EW // GB, chunk, 0)


_gather = pl.kernel(
    _gather_body,
    out_type=(
        jax.ShapeDtypeStruct((E // 4, 128), jnp.float32),
        jax.ShapeDtypeStruct((E // 4, 128), jnp.float32),
    ),
    mesh=_mesh,
    compiler_params=_sc_params,
    scratch_types=[
        pltpu.VMEM((GB,), jnp.int32),
        pltpu.VMEM((GB,), jnp.int32),
        pltpu.VMEM((GB, TW), jnp.float32),
        pltpu.VMEM((GB, TW), jnp.float32),
        pltpu.VMEM((GB // 4, 128), jnp.float32),
        pltpu.VMEM((GB // 4, 128), jnp.float32),
        pltpu.SemaphoreType.DMA,
        pltpu.SemaphoreType.DMA,
    ],
)


NSUB = N // NS
ZB = 250


def _scatter_body(d0_hbm, d1_hbm, i_hbm, out0_hbm, out1_hbm, ib, db64, db, zb,
                  table):
    cid = lax.axis_index("c")
    sid = lax.axis_index("s")

    def zrow(r, cc):
        zb[r, :] = jnp.zeros((16,), jnp.float32)
        return cc

    lax.fori_loop(0, ZB, zrow, 0)

    def zcp(k, cc):
        pltpu.sync_copy(zb, table.at[pl.ds(sid * NSUB + k * ZB, ZB)])
        return cc

    lax.fori_loop(0, NSUB // ZB, zcp, 0)
    plsc.subcore_barrier()

    def chunk(c, carry):
        e0 = sid * ES + c * SB
        pltpu.sync_copy(i_hbm.at[pl.ds(e0, SB)], ib)

        @pl.when(cid == 0)
        def _():
            pltpu.sync_copy(d0_hbm.at[pl.ds(e0 // 4, SB // 4)], db64)

        @pl.when(cid == 1)
        def _():
            pltpu.sync_copy(d1_hbm.at[pl.ds(e0 // 4, SB // 4)], db64)

        def rpk(q, cc):
            for r in range(4):
                db[4 * q + r, :] = db64[q, pl.ds(16 * r, 16)]
            return cc

        lax.fori_loop(0, SB // 4, rpk, 0)
        pltpu.sync_copy(db, table.at[ib], add=True)
        return carry

    lax.fori_loop(0, ES // SB, chunk, 0)
    plsc.subcore_barrier()

    def dump(k, cc):
        sl = pl.ds(sid * NSUB + k * ZB, ZB)

        @pl.when(cid == 0)
        def _():
            pltpu.sync_copy(table.at[sl], out0_hbm.at[sl])

        @pl.when(cid == 1)
        def _():
            pltpu.sync_copy(table.at[sl], out1_hbm.at[sl])

        return cc

    lax.fori_loop(0, NSUB // ZB, dump, 0)


_scatter = pl.kernel(
    _scatter_body,
    out_type=(
        jax.ShapeDtypeStruct((N, 16), jnp.float32),
        jax.ShapeDtypeStruct((N, 16), jnp.float32),
    ),
    mesh=_mesh,
    compiler_params=_sc_params,
    scratch_types=[
        pltpu.VMEM((SB,), jnp.int32),
        pltpu.VMEM((SB // 4, 64), jnp.float32),
        pltpu.VMEM((SB, 16), jnp.float32),
        pltpu.VMEM((ZB, 16), jnp.float32),
        pltpu.VMEM_SHARED((N, 16), jnp.float32),
    ],
)


_NB = 2000
_RB = 1600


def _full(i):
    return (0, 0)


def _rows(i):
    return (i, 0)


def _prep0_body(x_ref, pos_ref, wa_ref, wb_ref, b_ref, p_ref, q_ref):
    xb = x_ref[...]
    cb = pos_ref[...]
    z = jnp.zeros((xb.shape[0], TW - 34), jnp.float32)
    p = jnp.dot(xb, wa_ref[...], preferred_element_type=jnp.float32) + b_ref[...]
    q = jnp.dot(xb, wb_ref[...], preferred_element_type=jnp.float32)
    p_ref[...] = jnp.concatenate([p, cb, z], axis=1)
    q_ref[...] = jnp.concatenate([q, -cb, z], axis=1)


def _prep0(x, pos, wa, wb, b):
    return pl.pallas_call(
        _prep0_body,
        grid=(N // _NB,),
        in_specs=[
            pl.BlockSpec((_NB, 2), _rows),
            pl.BlockSpec((_NB, 2), _rows),
            pl.BlockSpec(wa.shape, _full),
            pl.BlockSpec(wb.shape, _full),
            pl.BlockSpec(b.shape, _full),
        ],
        out_specs=[
            pl.BlockSpec((_NB, TW), _rows),
            pl.BlockSpec((_NB, TW), _rows),
        ],
        out_shape=[
            jax.ShapeDtypeStruct((N, TW), jnp.float32),
            jax.ShapeDtypeStruct((N, TW), jnp.float32),
        ],
    )(x, pos, wa, wb, b)


def _edge_consts(p, f):
    f32 = jnp.float32
    i4 = jnp.eye(4, dtype=f32)
    wd = p["W_e1"][2 * f]
    wa = p["W_e1"][2 * f + 1]
    d = jnp.zeros((32, 32), f32).at[0].set(wd).at[1].set(wd)
    p2 = jnp.kron(i4, d)
    s = jnp.kron(i4, wa[None, :])
    w2 = jnp.kron(i4, p["W_e2"])
    b2 = jnp.tile(p["b_e2"], 4)[None]
    wc1 = jnp.kron(i4, p["W_c1"])
    bc1 = jnp.tile(p["b_c1"], 4)[None]
    wc2 = jnp.kron(i4, jnp.tile(p["W_c2"], (1, 16)))
    bc2 = jnp.full((1, 64), p["b_c2"][0], f32)
    r = jnp.zeros((32, 16), f32).at[0, 0].set(1.0).at[1, 1].set(1.0)
    rr = jnp.kron(i4, r)
    cc = jnp.tile(jnp.zeros((16,), f32).at[2].set(1.0), 4)[None]
    e16 = jnp.eye(16, dtype=f32)
    slo = jnp.kron(i4, jnp.concatenate([e16, jnp.zeros((16, 16), f32)]))
    shi = jnp.kron(i4, jnp.concatenate([jnp.zeros((16, 16), f32), e16]))
    return (p2, s, w2, b2, wc1, bc1, wc2, bc2, rr, cc, slo, shi)


def _edge_body(with_m, h_ref, rel_ref, ea_ref, p2_ref, s_ref, w2_ref, b2_ref,
               wc1_ref, bc1_ref, wc2_ref, bc2_ref, rr_ref, cc_ref, slo_ref,
               shi_ref, *out_refs):
    rel = rel_ref[...]
    sq = rel * rel

    def mm(a, b):
        return jnp.dot(a, b, preferred_element_type=jnp.float32)

    pre = h_ref[...] + mm(sq, p2_ref[...]) + mm(ea_ref[...], s_ref[...])
    m = jax.nn.silu(pre)
    m = jax.nn.silu(mm(m, w2_ref[...]) + b2_ref[...])
    c = jax.nn.silu(mm(m, wc1_ref[...]) + bc1_ref[...])
    cw = mm(c, wc2_ref[...]) + bc2_ref[...]
    rel64 = mm(rel, rr_ref[...])
    if with_m:
        out_refs[0][...] = mm(m, slo_ref[...])
        out_refs[1][...] = mm(m, shi_ref[...])
    out_refs[-1][...] = rel64 * cw + cc_ref[...]


def _edge(with_m, h128, rel128, ea4, consts):
    n_out = 3 if with_m else 1
    outs = [jax.ShapeDtypeStruct((E // 4, 64), jnp.float32)] * n_out
    ospecs = [pl.BlockSpec((_RB, 64), _rows)] * n_out
    return pl.pallas_call(
        functools.partial(_edge_body, with_m),
        grid=((E // 4) // _RB,),
        in_specs=[
            pl.BlockSpec((_RB, 128), _rows),
            pl.BlockSpec((_RB, 128), _rows),
            pl.BlockSpec((_RB, 4), _rows),
        ] + [pl.BlockSpec(w.shape, _full) for w in consts],
        out_specs=ospecs,
        out_shape=outs,
    )(h128, rel128, ea4, *consts)


def _node0_body(x_ref, mlo_ref, mhi_ref, ag_ref, pos_ref,
                wn1x_ref, wn1a_ref, wn1b_ref, bn1_ref, wn2_ref, bn2_ref,
                w1p_ref, w1q_ref, b1_ref, p_ref, q_ref):
    ag = ag_ref[...]
    num = ag[:, 0:2]
    cnt = ag[:, 2:3]
    coors1 = pos_ref[...] + num / jnp.maximum(cnt, 1.0)

    def mm(a, b):
        return jnp.dot(a, b, preferred_element_type=jnp.float32)

    h = jax.nn.silu(mm(x_ref[...], wn1x_ref[...]) + mm(mlo_ref[...], wn1a_ref[...])
                    + mm(mhi_ref[...], wn1b_ref[...]) + bn1_ref[...])
    feats1 = mm(h, wn2_ref[...]) + bn2_ref[...]
    z = jnp.zeros((feats1.shape[0], TW - 34), jnp.float32)
    p = mm(feats1, w1p_ref[...]) + b1_ref[...]
    q = mm(feats1, w1q_ref[...])
    p_ref[...] = jnp.concatenate([p, coors1, z], axis=1)
    q_ref[...] = jnp.concatenate([q, -coors1, z], axis=1)


def _node0(x, mlo, mhi, ag, pos, wn1x, wn1a, wn1b, bn1, wn2, bn2, w1p, w1q, b1):
    return pl.pallas_call(
        _node0_body,
        grid=(N // _NB,),
        in_specs=[
            pl.BlockSpec((_NB, 2), _rows),
            pl.BlockSpec((_NB, 16), _rows),
            pl.BlockSpec((_NB, 16), _rows),
            pl.BlockSpec((_NB, 16), _rows),
            pl.BlockSpec((_NB, 2), _rows),
        ] + [pl.BlockSpec(w.shape, _full)
             for w in (wn1x, wn1a, wn1b, bn1, wn2, bn2, w1p, w1q, b1)],
        out_specs=[
            pl.BlockSpec((_NB, TW), _rows),
            pl.BlockSpec((_NB, TW), _rows),
        ],
        out_shape=[
            jax.ShapeDtypeStruct((N, TW), jnp.float32),
            jax.ShapeDtypeStruct((N, TW), jnp.float32),
        ],
    )(x, mlo, mhi, ag, pos, wn1x, wn1a, wn1b, bn1, wn2, bn2, w1p, w1q, b1)


def _node1_body(p_ref, ag_ref, o_ref):
    ag = ag_ref[...]
    num = ag[:, 0:2]
    cnt = ag[:, 2:3]
    o_ref[...] = p_ref[:, 32:34] + num / jnp.maximum(cnt, 1.0)


def _node1(p1, ag):
    return pl.pallas_call(
        _node1_body,
        grid=(N // _NB,),
        in_specs=[
            pl.BlockSpec((_NB, TW), _rows),
            pl.BlockSpec((_NB, 16), _rows),
        ],
        out_specs=pl.BlockSpec((_NB, 2), _rows),
        out_shape=jax.ShapeDtypeStruct((N, 2), jnp.float32),
    )(p1, ag)


def kernel(x, edge_index, edge_attr, batch, positions, params):
    del batch
    x = x.astype(jnp.float32)
    i1 = edge_index[0]
    j1 = edge_index[1]
    ea4 = edge_attr.reshape(E // 4, 4)

    p0 = params["l0"]
    p1 = params["l1"]
    f0 = 2

    P, Q = _prep0(x, positions,
                  p0["W_e1"][0:f0], p0["W_e1"][f0:2 * f0],
                  p0["b_e1"].reshape(1, 32))
    h128, rel128 = _gather(P, Q, i1, j1)
    mlo64, mhi64, aux64 = _edge(True, h128, rel128, ea4, _edge_consts(p0, f0))
    milo, mihi = _scatter(mlo64, mhi64, i1)
    ag, _unused = _scatter(aux64, aux64, i1)
    P1, Q1 = _node0(
        x, milo, mihi, ag, positions,
        p0["W_n1"][0:2], p0["W_n1"][2:18], p0["W_n1"][18:34],
        p0["b_n1"].reshape(1, 32), p0["W_n2"], p0["b_n2"].reshape(1, 32),
        p1["W_e1"][0:32], p1["W_e1"][32:64], p1["b_e1"].reshape(1, 32))

    h1, rel1 = _gather(P1, Q1, i1, j1)
    (aux64_1,) = _edge(False, h1, rel1, ea4, _edge_consts(p1, 32))
    ag1, _unused1 = _scatter(aux64_1, aux64_1, i1)
    return _node1(P1, ag1)


# speedup vs baseline: 1.1254x; 1.1254x over previous
import functools

import jax
import jax.numpy as jnp
from jax import lax
from jax.experimental import pallas as pl
from jax.experimental.pallas import tpu as pltpu
from jax.experimental.pallas import tpu_sc as plsc

N = 100000
E = 3200000
NC, NS = 2, 16
NW = NC * NS
TW = 48
GB = 800
SB = 800
EW = E // NW
ES = E // NS

_mesh = plsc.VectorSubcoreMesh(core_axis_name="c", subcore_axis_name="s")
_sc_params = pltpu.CompilerParams(use_tc_tiling_on_sc=False)


def _gather_body(p_hbm, q_hbm, i_hbm, j_hbm, out_hbm, ib, jb, pb, qb, s1, s2):
    cid = lax.axis_index("c")
    sid = lax.axis_index("s")
    w = sid * NC + cid

    def chunk(c, carry):
        e0 = w * EW + c * GB
        pltpu.sync_copy(i_hbm.at[pl.ds(e0, GB)], ib)
        pltpu.sync_copy(j_hbm.at[pl.ds(e0, GB)], jb)
        cp1 = pltpu.async_copy(p_hbm.at[ib], pb, s1)
        cp2 = pltpu.async_copy(q_hbm.at[jb], qb, s2)
        cp1.wait()
        cp2.wait()

        def add_row(e, cc):
            for g in range(TW // 16):
                sl = pl.ds(16 * g, 16)
                pb[e, sl] = pb[e, sl] + qb[e, sl]
            return cc

        lax.fori_loop(0, GB, add_row, 0)
        pltpu.sync_copy(pb, out_hbm.at[pl.ds(e0, GB)])
        return carry

    lax.fori_loop(0, EW // GB, chunk, 0)


_gather = pl.kernel(
    _gather_body,
    out_type=jax.ShapeDtypeStruct((E, TW), jnp.float32),
    mesh=_mesh,
    compiler_params=_sc_params,
    scratch_types=[
        pltpu.VMEM((GB,), jnp.int32),
        pltpu.VMEM((GB,), jnp.int32),
        pltpu.VMEM((GB, TW), jnp.float32),
        pltpu.VMEM((GB, TW), jnp.float32),
        pltpu.SemaphoreType.DMA,
        pltpu.SemaphoreType.DMA,
    ],
)


NSUB = N // NS
ZB = 625


def _scatter_body(d0_hbm, d1_hbm, i_hbm, out0_hbm, out1_hbm, ib, db, zb, table):
    cid = lax.axis_index("c")
    sid = lax.axis_index("s")

    def zrow(r, cc):
        zb[r, :] = jnp.zeros((16,), jnp.float32)
        return cc

    lax.fori_loop(0, ZB, zrow, 0)

    def zcp(k, cc):
        pltpu.sync_copy(zb, table.at[pl.ds(sid * NSUB + k * ZB, ZB)])
        return cc

    lax.fori_loop(0, NSUB // ZB, zcp, 0)
    plsc.subcore_barrier()

    def chunk(c, carry):
        e0 = sid * ES + c * SB
        pltpu.sync_copy(i_hbm.at[pl.ds(e0, SB)], ib)

        @pl.when(cid == 0)
        def _():
            pltpu.sync_copy(d0_hbm.at[pl.ds(e0, SB)], db)

        @pl.when(cid == 1)
        def _():
            pltpu.sync_copy(d1_hbm.at[pl.ds(e0, SB)], db)

        pltpu.sync_copy(db, table.at[ib], add=True)
        return carry

    lax.fori_loop(0, ES // SB, chunk, 0)
    plsc.subcore_barrier()

    def dump(k, cc):
        sl = pl.ds(sid * NSUB + k * ZB, ZB)

        @pl.when(cid == 0)
        def _():
            pltpu.sync_copy(table.at[sl], out0_hbm.at[sl])

        @pl.when(cid == 1)
        def _():
            pltpu.sync_copy(table.at[sl], out1_hbm.at[sl])

        return cc

    lax.fori_loop(0, NSUB // ZB, dump, 0)


_scatter = pl.kernel(
    _scatter_body,
    out_type=(
        jax.ShapeDtypeStruct((N, 16), jnp.float32),
        jax.ShapeDtypeStruct((N, 16), jnp.float32),
    ),
    mesh=_mesh,
    compiler_params=_sc_params,
    scratch_types=[
        pltpu.VMEM((SB,), jnp.int32),
        pltpu.VMEM((SB, 16), jnp.float32),
        pltpu.VMEM((ZB, 16), jnp.float32),
        pltpu.VMEM_SHARED((N, 16), jnp.float32),
    ],
)


_NB = 2000
_RB = 1600


def _full(i):
    return (0, 0)


def _rows(i):
    return (i, 0)


def _prep0_body(x_ref, pos_ref, wa_ref, wb_ref, b_ref, p_ref, q_ref):
    xb = x_ref[...]
    cb = pos_ref[...]
    z = jnp.zeros((xb.shape[0], TW - 34), jnp.float32)
    p = jnp.dot(xb, wa_ref[...], preferred_element_type=jnp.float32) + b_ref[...]
    q = jnp.dot(xb, wb_ref[...], preferred_element_type=jnp.float32)
    p_ref[...] = jnp.concatenate([p, cb, z], axis=1)
    q_ref[...] = jnp.concatenate([q, -cb, z], axis=1)


def _prep0(x, pos, wa, wb, b):
    return pl.pallas_call(
        _prep0_body,
        grid=(N // _NB,),
        in_specs=[
            pl.BlockSpec((_NB, 2), _rows),
            pl.BlockSpec((_NB, 2), _rows),
            pl.BlockSpec(wa.shape, _full),
            pl.BlockSpec(wb.shape, _full),
            pl.BlockSpec(b.shape, _full),
        ],
        out_specs=[
            pl.BlockSpec((_NB, TW), _rows),
            pl.BlockSpec((_NB, TW), _rows),
        ],
        out_shape=[
            jax.ShapeDtypeStruct((N, TW), jnp.float32),
            jax.ShapeDtypeStruct((N, TW), jnp.float32),
        ],
    )(x, pos, wa, wb, b)


def _edge_consts(p, f):
    f32 = jnp.float32
    i4 = jnp.eye(4, dtype=f32)
    wd = p["W_e1"][2 * f]
    wa = p["W_e1"][2 * f + 1]
    sel = jnp.zeros((TW, 32), f32).at[:32].set(jnp.eye(32, dtype=f32))
    p1 = jnp.kron(i4, sel)
    d = jnp.zeros((TW, 32), f32).at[32].set(wd).at[33].set(wd)
    p2 = jnp.kron(i4, d)
    s = jnp.kron(i4, wa[None, :])
    w2 = jnp.kron(i4, p["W_e2"])
    b2 = jnp.tile(p["b_e2"], 4)[None]
    wc1 = jnp.kron(i4, p["W_c1"])
    bc1 = jnp.tile(p["b_c1"], 4)[None]
    wc2 = jnp.kron(i4, jnp.tile(p["W_c2"], (1, 16)))
    bc2 = jnp.full((1, 64), p["b_c2"][0], f32)
    r = jnp.zeros((TW, 16), f32).at[32, 0].set(1.0).at[33, 1].set(1.0)
    rr = jnp.kron(i4, r)
    cc = jnp.tile(jnp.zeros((16,), f32).at[2].set(1.0), 4)[None]
    e16 = jnp.eye(16, dtype=f32)
    slo = jnp.kron(i4, jnp.concatenate([e16, jnp.zeros((16, 16), f32)]))
    shi = jnp.kron(i4, jnp.concatenate([jnp.zeros((16, 16), f32), e16]))
    return (p1, p2, s, w2, b2, wc1, bc1, wc2, bc2, rr, cc, slo, shi)


def _edge_body(with_m, hs_ref, ea_ref, p1_ref, p2_ref, s_ref, w2_ref, b2_ref,
               wc1_ref, bc1_ref, wc2_ref, bc2_ref, rr_ref, cc_ref, slo_ref,
               shi_ref, *out_refs):
    hs = hs_ref[...]
    sq = hs * hs

    def mm(a, b):
        return jnp.dot(a, b, preferred_element_type=jnp.float32)

    pre = mm(hs, p1_ref[...]) + mm(sq, p2_ref[...]) + mm(ea_ref[...], s_ref[...])
    m = jax.nn.silu(pre)
    m = jax.nn.silu(mm(m, w2_ref[...]) + b2_ref[...])
    c = jax.nn.silu(mm(m, wc1_ref[...]) + bc1_ref[...])
    cw = mm(c, wc2_ref[...]) + bc2_ref[...]
    rel = mm(hs, rr_ref[...])
    if with_m:
        out_refs[0][...] = mm(m, slo_ref[...])
        out_refs[1][...] = mm(m, shi_ref[...])
    out_refs[-1][...] = rel * cw + cc_ref[...]


def _edge(with_m, hs192, ea4, consts):
    n_out = 3 if with_m else 1
    outs = [jax.ShapeDtypeStruct((E // 4, 64), jnp.float32)] * n_out
    ospecs = [pl.BlockSpec((_RB, 64), _rows)] * n_out
    return pl.pallas_call(
        functools.partial(_edge_body, with_m),
        grid=((E // 4) // _RB,),
        in_specs=[
            pl.BlockSpec((_RB, 192), _rows),
            pl.BlockSpec((_RB, 4), _rows),
        ] + [pl.BlockSpec(w.shape, _full) for w in consts],
        out_specs=ospecs,
        out_shape=outs,
    )(hs192, ea4, *consts)


def _node0_body(x_ref, mlo_ref, mhi_ref, ag_ref, pos_ref,
                wn1x_ref, wn1a_ref, wn1b_ref, bn1_ref, wn2_ref, bn2_ref,
                w1p_ref, w1q_ref, b1_ref, p_ref, q_ref):
    ag = ag_ref[...]
    num = ag[:, 0:2]
    cnt = ag[:, 2:3]
    coors1 = pos_ref[...] + num / jnp.maximum(cnt, 1.0)

    def mm(a, b):
        return jnp.dot(a, b, preferred_element_type=jnp.float32)

    h = jax.nn.silu(mm(x_ref[...], wn1x_ref[...]) + mm(mlo_ref[...], wn1a_ref[...])
                    + mm(mhi_ref[...], wn1b_ref[...]) + bn1_ref[...])
    feats1 = mm(h, wn2_ref[...]) + bn2_ref[...]
    z = jnp.zeros((feats1.shape[0], TW - 34), jnp.float32)
    p = mm(feats1, w1p_ref[...]) + b1_ref[...]
    q = mm(feats1, w1q_ref[...])
    p_ref[...] = jnp.concatenate([p, coors1, z], axis=1)
    q_ref[...] = jnp.concatenate([q, -coors1, z], axis=1)


def _node0(x, mlo, mhi, ag, pos, wn1x, wn1a, wn1b, bn1, wn2, bn2, w1p, w1q, b1):
    return pl.pallas_call(
        _node0_body,
        grid=(N // _NB,),
        in_specs=[
            pl.BlockSpec((_NB, 2), _rows),
            pl.BlockSpec((_NB, 16), _rows),
            pl.BlockSpec((_NB, 16), _rows),
            pl.BlockSpec((_NB, 16), _rows),
            pl.BlockSpec((_NB, 2), _rows),
        ] + [pl.BlockSpec(w.shape, _full)
             for w in (wn1x, wn1a, wn1b, bn1, wn2, bn2, w1p, w1q, b1)],
        out_specs=[
            pl.BlockSpec((_NB, TW), _rows),
            pl.BlockSpec((_NB, TW), _rows),
        ],
        out_shape=[
            jax.ShapeDtypeStruct((N, TW), jnp.float32),
            jax.ShapeDtypeStruct((N, TW), jnp.float32),
        ],
    )(x, mlo, mhi, ag, pos, wn1x, wn1a, wn1b, bn1, wn2, bn2, w1p, w1q, b1)


def _node1_body(p_ref, ag_ref, o_ref):
    ag = ag_ref[...]
    num = ag[:, 0:2]
    cnt = ag[:, 2:3]
    o_ref[...] = p_ref[:, 32:34] + num / jnp.maximum(cnt, 1.0)


def _node1(p1, ag):
    return pl.pallas_call(
        _node1_body,
        grid=(N // _NB,),
        in_specs=[
            pl.BlockSpec((_NB, TW), _rows),
            pl.BlockSpec((_NB, 16), _rows),
        ],
        out_specs=pl.BlockSpec((_NB, 2), _rows),
        out_shape=jax.ShapeDtypeStruct((N, 2), jnp.float32),
    )(p1, ag)


def kernel(x, edge_index, edge_attr, batch, positions, params):
    del batch
    x = x.astype(jnp.float32)
    i1 = edge_index[0]
    j1 = edge_index[1]
    ea4 = edge_attr.reshape(E // 4, 4)

    p0 = params["l0"]
    p1 = params["l1"]
    f0 = 2

    P, Q = _prep0(x, positions,
                  p0["W_e1"][0:f0], p0["W_e1"][f0:2 * f0],
                  p0["b_e1"].reshape(1, 32))
    hs = _gather(P, Q, i1, j1).reshape(E // 4, 4 * TW)
    mlo64, mhi64, aux64 = _edge(True, hs, ea4, _edge_consts(p0, f0))
    milo, mihi = _scatter(mlo64.reshape(E, 16), mhi64.reshape(E, 16), i1)
    aux = aux64.reshape(E, 16)
    ag, _unused = _scatter(aux, aux, i1)
    P1, Q1 = _node0(
        x, milo, mihi, ag, positions,
        p0["W_n1"][0:2], p0["W_n1"][2:18], p0["W_n1"][18:34],
        p0["b_n1"].reshape(1, 32), p0["W_n2"], p0["b_n2"].reshape(1, 32),
        p1["W_e1"][0:32], p1["W_e1"][32:64], p1["b_e1"].reshape(1, 32))

    hs1 = _gather(P1, Q1, i1, j1).reshape(E // 4, 4 * TW)
    (aux64_1,) = _edge(False, hs1, ea4, _edge_consts(p1, 32))
    aux1 = aux64_1.reshape(E, 16)
    ag1, _unused1 = _scatter(aux1, aux1, i1)
    return _node1(P1, ag1)
